# baseline (speedup 1.0000x reference)
import functools

import jax
import jax.numpy as jnp
from jax import lax
from jax.experimental import pallas as pl
from jax.experimental.pallas import tpu as pltpu
from jax.experimental.pallas import tpu_sc as plsc

_NEG = float(jnp.finfo(jnp.float32).min)
_NC, _NS, _L = 2, 16, 16


def _elu(v):
    return jnp.where(v > 0, v, jnp.exp(jnp.minimum(v, 0.0)) - 1.0)


def _build_body(N, src2_hbm, dst2_hbm, za_hbm, A_hbm,
                srcA, dstA, srcB, dstB, Abuf,
                sem_sa, sem_da, sem_sb, sem_db):
    c = lax.axis_index("c")
    s = lax.axis_index("s")
    wid = s * _NC + c
    one_vec = jnp.full((_L,), 1.0, jnp.float32)
    zero_vec = jnp.full((_L,), 0.0, jnp.float32)
    zero_ivec = jnp.full((_L,), 0, jnp.int32)
    r32u = jnp.full((_L,), 32, jnp.uint32)

    bufs = ((srcA, dstA, sem_sa, sem_da), (srcB, dstB, sem_sb, sem_db))
    steps = [(p, ch) for p in range(2) for ch in range(4)]

    def start(i):
        sb, db, ss, ds_ = bufs[i % 2]
        ch = steps[i][1]
        return (pltpu.async_copy(src2_hbm.at[pl.ds(ch * 64, 64)], sb, ss),
                pltpu.async_copy(dst2_hbm.at[pl.ds(ch * 64, 64)], db, ds_))

    pending = {0: start(0)}
    for i, (p, ch) in enumerate(steps):
        if ch == 0:
            row0 = wid * 64 + p * 32
            lo_vec = jnp.full((_L,), row0, jnp.int32)
            pltpu.sync_copy(za_hbm, Abuf)
        if i + 1 < len(steps):
            pending[i + 1] = start(i + 1)
        for d in pending.pop(i):
            d.wait()
        sb, db = bufs[i % 2][0], bufs[i % 2][1]

        def scanrow(r, carry):
            for q in range(8):
                sv = sb[r, pl.ds(q * 16, 16)]
                dv = db[r, pl.ds(q * 16, 16)]
                u = dv - lo_vec
                m = plsc.bitcast(u, jnp.uint32) < r32u
                rowi = jnp.where(m, u, zero_ivec)
                val = jnp.where(m, one_vec, zero_vec)
                plsc.addupdate_scatter(Abuf, [rowi, sv], val)
            return carry

        lax.fori_loop(0, 64, scanrow, 0)
        if ch == 3:
            pltpu.sync_copy(Abuf, A_hbm.at[pl.ds(row0, 32)])


def _sc_build(src2, dst2, za):
    N = 2048
    mesh = plsc.VectorSubcoreMesh(core_axis_name="c", subcore_axis_name="s",
                                  num_cores=_NC, num_subcores=_NS)
    fn = pl.kernel(
        functools.partial(_build_body, N),
        out_type=jax.ShapeDtypeStruct((N, N), jnp.float32),
        mesh=mesh,
        compiler_params=pltpu.CompilerParams(needs_layout_passes=False),
        scratch_types=[
            pltpu.VMEM((64, 128), jnp.int32),
            pltpu.VMEM((64, 128), jnp.int32),
            pltpu.VMEM((64, 128), jnp.int32),
            pltpu.VMEM((64, 128), jnp.int32),
            pltpu.VMEM((32, 2048), jnp.float32),
            pltpu.SemaphoreType.DMA,
            pltpu.SemaphoreType.DMA,
            pltpu.SemaphoreType.DMA,
            pltpu.SemaphoreType.DMA,
        ],
    )
    return fn(src2, dst2, za)


def _mj_body(A_ref, J_ref, mj_ref, ahi_ref, alo_ref):
    mj_ref[:, 0, :] = jnp.dot(J_ref[...], A_ref[...],
                              preferred_element_type=jnp.float32,
                              precision=lax.Precision.HIGHEST)
    mj_ref[:, 1, :] = J_ref[...]
    a = A_ref[...]
    ahi = a.astype(jnp.bfloat16)
    ahi_ref[...] = ahi
    alo_ref[...] = (a - ahi.astype(jnp.float32)).astype(jnp.bfloat16)


def _tc_mj(A, J):
    B, N = J.shape
    return pl.pallas_call(
        _mj_body,
        out_shape=(jax.ShapeDtypeStruct((B, 2, N), jnp.float32),
                   jax.ShapeDtypeStruct((N, N), jnp.bfloat16),
                   jax.ShapeDtypeStruct((N, N), jnp.bfloat16)),
        compiler_params=pltpu.CompilerParams(
            vmem_limit_bytes=56 * 1024 * 1024),
    )(A, J)


def _mm_body(ahi_ref, alo_ref, x_ref, MJ_ref, W1_ref, W1s_ref, b1_ref,
             out_ref, h_scr):
    x = x_ref[...]
    xhi = x.astype(jnp.bfloat16)
    xlo = (x - xhi.astype(jnp.float32)).astype(jnp.bfloat16)
    agg = jnp.dot(ahi_ref[...], xhi, preferred_element_type=jnp.float32)
    agg += jnp.dot(ahi_ref[...], xlo, preferred_element_type=jnp.float32)
    agg += jnp.dot(alo_ref[...], xhi, preferred_element_type=jnp.float32)
    acc = jnp.dot(agg, W1_ref[...], preferred_element_type=jnp.float32)
    acc += jnp.dot(x, W1s_ref[...], preferred_element_type=jnp.float32)
    h_scr[...] = _elu(acc + b1_ref[...])
    out_ref[0] = jnp.dot(MJ_ref[0], h_scr[...],
                         preferred_element_type=jnp.float32)


def _tc_mm(Ahi, Alo, x_flat, MJ, W1, W1s, b1):
    BN, F = x_flat.shape
    H = W1.shape[1]
    B = MJ.shape[0]
    blk = BN // B
    return pl.pallas_call(
        _mm_body,
        grid=(B,),
        in_specs=[
            pl.BlockSpec((blk, blk), lambda b: (0, 0)),
            pl.BlockSpec((blk, blk), lambda b: (0, 0)),
            pl.BlockSpec((blk, F), lambda b: (b, 0)),
            pl.BlockSpec((1, 2, blk), lambda b: (b, 0, 0)),
            pl.BlockSpec((F, H), lambda b: (0, 0)),
            pl.BlockSpec((F, H), lambda b: (0, 0)),
            pl.BlockSpec((1, H), lambda b: (0, 0)),
        ],
        out_specs=pl.BlockSpec((1, 2, H), lambda b: (b, 0, 0)),
        out_shape=jax.ShapeDtypeStruct((B, 2, H), jnp.float32),
        scratch_shapes=[pltpu.VMEM((blk, H), jnp.float32)],
        compiler_params=pltpu.CompilerParams(
            vmem_limit_bytes=56 * 1024 * 1024),
    )(Ahi, Alo, x_flat, MJ, W1, W1s, b1[None, :])


def _head_body(rh_ref, W2_ref, W2s_ref, b2_ref, p3_ref, mask_ref,
               gum_ref, Wae_ref, Wap_ref, ba_ref, Wb_ref, bb_ref, Wc_ref,
               bc_ref, act_ref, lm_ref):
    r2 = rh_ref[:, 0, :]
    hj = rh_ref[:, 1, :]
    out_rows = _elu(
        jnp.dot(r2, W2_ref[...], preferred_element_type=jnp.float32)
        + jnp.dot(hj, W2s_ref[...], preferred_element_type=jnp.float32)
        + b2_ref[...])
    h = _elu(
        jnp.dot(out_rows, Wae_ref[...], preferred_element_type=jnp.float32)
        + jnp.dot(p3_ref[...], Wap_ref[...], preferred_element_type=jnp.float32)
        + ba_ref[...])
    h = _elu(jnp.dot(h, Wb_ref[...], preferred_element_type=jnp.float32)
             + bb_ref[...])
    logits = (jnp.dot(h, Wc_ref[...], preferred_element_type=jnp.float32)
              + bc_ref[...])
    lm = jnp.where(mask_ref[...] != 0, logits, _NEG)
    lm_ref[...] = lm
    act_ref[...] = jnp.argmax(lm + gum_ref[...], axis=-1).astype(jnp.int32)[None, :]


def _tc_head(rh, W2, W2s, b2, p3, mask, gumbel, Wae, Wap, ba, Wb, bb, Wc, bc):
    B = rh.shape[0]
    NA = Wc.shape[1]
    act2d, lm = pl.pallas_call(
        _head_body,
        out_shape=(jax.ShapeDtypeStruct((1, B), jnp.int32),
                   jax.ShapeDtypeStruct((B, NA), jnp.float32)),
    )(rh, W2, W2s, b2[None, :], p3, mask, gumbel,
      Wae, Wap, ba[None, :], Wb, bb[None, :], Wc, bc[None, :])
    return act2d[0], lm


def kernel(map_tensor, piece_tensor, edge_index, W1, W1s, b1, W2, W2s, b2,
           Wa, ba, Wb, bb, Wc, bc):
    B = map_tensor.shape[0]
    F = map_tensor.shape[2]
    x = map_tensor.reshape(B, -1, F)
    N = x.shape[1]
    x_flat = x.reshape(B * N, F)
    E = edge_index.shape[1]

    src = edge_index[0].astype(jnp.int32)
    dst = edge_index[1].astype(jnp.int32)
    src2 = src.reshape(E // 128, 128)
    dst2 = dst.reshape(E // 128, 128)
    za = jnp.zeros((32, N), jnp.float32)

    p_type = piece_tensor[:, 0].astype(jnp.int32)
    pos = piece_tensor[:, 1:3].astype(jnp.int32)
    action_mask = piece_tensor[:, 3:16].astype(jnp.int32)
    j = pos[:, 0] * 12 + pos[:, 1]
    p3 = jax.nn.one_hot(p_type, 3, dtype=jnp.float32)
    gumbel = jax.random.gumbel(jax.random.key(42), (B, Wc.shape[1]),
                               jnp.float32)

    A = _sc_build(src2, dst2, za)
    J = jax.nn.one_hot(j, N, dtype=jnp.float32)
    MJ, Ahi, Alo = _tc_mj(A, J)
    rh = _tc_mm(Ahi, Alo, x_flat, MJ, W1, W1s, b1)
    action, lm = _tc_head(rh, W2, W2s, b2, p3, action_mask, gumbel,
                          Wa[:-3], Wa[-3:], ba, Wb, bb, Wc, bc)
    return (action, lm)


# speedup vs baseline: 1.5966x; 1.5966x over previous
import functools

import jax
import jax.numpy as jnp
from jax import lax
from jax.experimental import pallas as pl
from jax.experimental.pallas import tpu as pltpu
from jax.experimental.pallas import tpu_sc as plsc

_NEG = float(jnp.finfo(jnp.float32).min)
_NC, _NS, _L = 2, 16, 16


def _elu(v):
    return jnp.where(v > 0, v, jnp.exp(jnp.minimum(v, 0.0)) - 1.0)


def _build_body(N, src2_hbm, dst2_hbm, za_hbm, A_hbm,
                srcA, dstA, srcB, dstB, Abuf,
                sem_sa, sem_da, sem_sb, sem_db):
    c = lax.axis_index("c")
    s = lax.axis_index("s")
    wid = s * _NC + c
    one_vec = jnp.full((_L,), 1.0, jnp.float32)
    zero_vec = jnp.full((_L,), 0.0, jnp.float32)
    zero_ivec = jnp.full((_L,), 0, jnp.int32)
    r32u = jnp.full((_L,), 32, jnp.uint32)

    bufs = ((srcA, dstA, sem_sa, sem_da), (srcB, dstB, sem_sb, sem_db))
    steps = [(p, ch) for p in range(2) for ch in range(4)]

    def start(i):
        sb, db, ss, ds_ = bufs[i % 2]
        ch = steps[i][1]
        return (pltpu.async_copy(src2_hbm.at[pl.ds(ch * 64, 64)], sb, ss),
                pltpu.async_copy(dst2_hbm.at[pl.ds(ch * 64, 64)], db, ds_))

    pending = {0: start(0)}
    for i, (p, ch) in enumerate(steps):
        if ch == 0:
            row0 = wid * 64 + p * 32
            lo_vec = jnp.full((_L,), row0, jnp.int32)
            pltpu.sync_copy(za_hbm, Abuf)
        if i + 1 < len(steps):
            pending[i + 1] = start(i + 1)
        for d in pending.pop(i):
            d.wait()
        sb, db = bufs[i % 2][0], bufs[i % 2][1]

        def scanrow(r, carry):
            for q in range(8):
                sv = sb[r, pl.ds(q * 16, 16)]
                dv = db[r, pl.ds(q * 16, 16)]
                u = dv - lo_vec
                m = plsc.bitcast(u, jnp.uint32) < r32u
                rowi = jnp.where(m, u, zero_ivec)
                val = jnp.where(m, one_vec, zero_vec)
                plsc.addupdate_scatter(Abuf, [rowi, sv], val)
            return carry

        lax.fori_loop(0, 64, scanrow, 0)
        if ch == 3:
            pltpu.sync_copy(Abuf, A_hbm.at[pl.ds(row0, 32)])


def _sc_build(src2, dst2, za):
    N = 2048
    mesh = plsc.VectorSubcoreMesh(core_axis_name="c", subcore_axis_name="s",
                                  num_cores=_NC, num_subcores=_NS)
    fn = pl.kernel(
        functools.partial(_build_body, N),
        out_type=jax.ShapeDtypeStruct((N, N), jnp.float32),
        mesh=mesh,
        compiler_params=pltpu.CompilerParams(needs_layout_passes=False),
        scratch_types=[
            pltpu.VMEM((64, 128), jnp.int32),
            pltpu.VMEM((64, 128), jnp.int32),
            pltpu.VMEM((64, 128), jnp.int32),
            pltpu.VMEM((64, 128), jnp.int32),
            pltpu.VMEM((32, 2048), jnp.float32),
            pltpu.SemaphoreType.DMA,
            pltpu.SemaphoreType.DMA,
            pltpu.SemaphoreType.DMA,
            pltpu.SemaphoreType.DMA,
        ],
    )
    return fn(src2, dst2, za)


def _mm_body(A_ref, x_ref, J_ref, W1_ref, W1s_ref, b1_ref,
             W2_ref, W2s_ref, b2_ref, p3_ref, mask_ref, gum_ref,
             Wae_ref, Wap_ref, ba_ref, Wb_ref, bb_ref, Wc_ref, bc_ref,
             act_ref, lm_ref, h_scr, mj_scr, rh_scr):
    b = pl.program_id(0)
    B = pl.num_programs(0)

    @pl.when(b == 0)
    def _():
        mj_scr[:, 0, :] = jnp.dot(J_ref[...], A_ref[...],
                                  preferred_element_type=jnp.float32)
        mj_scr[:, 1, :] = J_ref[...]

    agg = jnp.dot(A_ref[...], x_ref[...], preferred_element_type=jnp.float32)
    acc = jnp.dot(agg, W1_ref[...], preferred_element_type=jnp.float32)
    acc += jnp.dot(x_ref[...], W1s_ref[...],
                   preferred_element_type=jnp.float32)
    h_scr[...] = _elu(acc + b1_ref[...])
    rh_scr[b] = jnp.dot(mj_scr[b], h_scr[...],
                        preferred_element_type=jnp.float32)

    @pl.when(b == B - 1)
    def _():
        r2 = rh_scr[:, 0, :]
        hj = rh_scr[:, 1, :]
        out_rows = _elu(
            jnp.dot(r2, W2_ref[...], preferred_element_type=jnp.float32)
            + jnp.dot(hj, W2s_ref[...], preferred_element_type=jnp.float32)
            + b2_ref[...])
        h = _elu(
            jnp.dot(out_rows, Wae_ref[...], preferred_element_type=jnp.float32)
            + jnp.dot(p3_ref[...], Wap_ref[...],
                      preferred_element_type=jnp.float32)
            + ba_ref[...])
        h = _elu(jnp.dot(h, Wb_ref[...], preferred_element_type=jnp.float32)
                 + bb_ref[...])
        logits = (jnp.dot(h, Wc_ref[...], preferred_element_type=jnp.float32)
                  + bc_ref[...])
        lm = jnp.where(mask_ref[...] != 0, logits, _NEG)
        lm_ref[...] = lm
        act_ref[...] = jnp.argmax(
            lm + gum_ref[...], axis=-1).astype(jnp.int32)[None, :]


def _tc_mm(A, x_flat, J, W1, W1s, b1, W2, W2s, b2, p3, mask, gumbel,
           Wae, Wap, ba, Wb, bb, Wc, bc):
    BN, F = x_flat.shape
    H = W1.shape[1]
    B, N = J.shape
    NA = Wc.shape[1]
    blk = BN // B
    const = lambda shape: pl.BlockSpec(shape, lambda b: tuple(
        0 for _ in shape))
    return pl.pallas_call(
        _mm_body,
        grid=(B,),
        in_specs=[
            pl.BlockSpec((blk, blk), lambda b: (0, 0)),
            pl.BlockSpec((blk, F), lambda b: (b, 0)),
            const((B, N)),
            const((F, H)), const((F, H)), const((1, H)),
            const(W2.shape), const(W2s.shape), const((1, H)),
            const(p3.shape), const(mask.shape), const(gumbel.shape),
            const(Wae.shape), const(Wap.shape), const((1, Wae.shape[1])),
            const(Wb.shape), const((1, Wb.shape[1])),
            const(Wc.shape), const((1, NA)),
        ],
        out_specs=(pl.BlockSpec((1, B), lambda b: (0, 0)),
                   pl.BlockSpec((B, NA), lambda b: (0, 0))),
        out_shape=(jax.ShapeDtypeStruct((1, B), jnp.int32),
                   jax.ShapeDtypeStruct((B, NA), jnp.float32)),
        scratch_shapes=[pltpu.VMEM((blk, H), jnp.float32),
                        pltpu.VMEM((B, 2, N), jnp.float32),
                        pltpu.VMEM((B, 2, H), jnp.float32)],
        compiler_params=pltpu.CompilerParams(
            vmem_limit_bytes=56 * 1024 * 1024),
    )(A, x_flat, J, W1, W1s, b1[None, :], W2, W2s, b2[None, :], p3,
      mask, gumbel, Wae, Wap, ba[None, :], Wb, bb[None, :], Wc, bc[None, :])


def kernel(map_tensor, piece_tensor, edge_index, W1, W1s, b1, W2, W2s, b2,
           Wa, ba, Wb, bb, Wc, bc):
    B = map_tensor.shape[0]
    F = map_tensor.shape[2]
    x = map_tensor.reshape(B, -1, F)
    N = x.shape[1]
    x_flat = x.reshape(B * N, F)
    E = edge_index.shape[1]

    src = edge_index[0].astype(jnp.int32)
    dst = edge_index[1].astype(jnp.int32)
    src2 = src.reshape(E // 128, 128)
    dst2 = dst.reshape(E // 128, 128)
    za = jnp.zeros((32, N), jnp.float32)

    p_type = piece_tensor[:, 0].astype(jnp.int32)
    pos = piece_tensor[:, 1:3].astype(jnp.int32)
    action_mask = piece_tensor[:, 3:16].astype(jnp.int32)
    j = pos[:, 0] * 12 + pos[:, 1]
    p3 = jax.nn.one_hot(p_type, 3, dtype=jnp.float32)
    gumbel = jax.random.gumbel(jax.random.key(42), (B, Wc.shape[1]),
                               jnp.float32)

    A = _sc_build(src2, dst2, za)
    J = jax.nn.one_hot(j, N, dtype=jnp.float32)
    act2d, lm = _tc_mm(A, x_flat, J, W1, W1s, b1, W2, W2s, b2, p3,
                       action_mask, gumbel, Wa[:-3], Wa[-3:], ba, Wb, bb,
                       Wc, bc)
    return (act2d[0], lm)
